# Initial kernel scaffold; baseline (speedup 1.0000x reference)
import functools

import jax
import jax.numpy as jnp
from jax import lax
from jax.experimental import pallas as pl
from jax.experimental.pallas import tpu as pltpu
from jax.experimental.pallas import tpu_sc as plsc

_NUM_ROWS = 16384 * 26
_D = 16
_NC = 2
_NS = 16
_NW = _NC * _NS
_B_PER_W = _NUM_ROWS // _NW
_CHUNK = 1024
_NCHUNK = _B_PER_W // _CHUNK

_mesh = plsc.VectorSubcoreMesh(core_axis_name="c", subcore_axis_name="s")


@functools.partial(
    pl.kernel,
    mesh=_mesh,
    out_type=jax.ShapeDtypeStruct((_NUM_ROWS, _D), jnp.float32),
    scratch_types=[
        pltpu.VMEM((_CHUNK,), jnp.int32),
        pltpu.VMEM((_CHUNK, _D), jnp.float32),
        pltpu.SemaphoreType.DMA,
    ],
)
def _gather_rows(idx_hbm, table_hbm, out_hbm, idx_v, rows_v, sem):
    wid = lax.axis_index("s") * _NC + lax.axis_index("c")
    base = wid * _B_PER_W

    def body(i, _):
        off = base + i * _CHUNK
        pltpu.sync_copy(idx_hbm.at[pl.ds(off, _CHUNK)], idx_v)
        pltpu.async_copy(table_hbm.at[idx_v], rows_v, sem).wait()
        pltpu.sync_copy(rows_v, out_hbm.at[pl.ds(off, _CHUNK)])
        return _

    lax.fori_loop(0, _NCHUNK, body, 0)


def kernel(x, table):
    flat_idx = x.reshape(-1)
    out = _gather_rows(flat_idx, table)
    return out.reshape(x.shape[0], -1)


# speedup vs baseline: 1.1558x; 1.1558x over previous
import functools

import jax
import jax.numpy as jnp
from jax import lax
from jax.experimental import pallas as pl
from jax.experimental.pallas import tpu as pltpu
from jax.experimental.pallas import tpu_sc as plsc

_NUM_ROWS = 16384 * 26
_D = 16
_NC = 2
_NS = 16
_NW = _NC * _NS
_B_PER_W = _NUM_ROWS // _NW
_CHUNK = 1024
_NCHUNK = _B_PER_W // _CHUNK

_mesh = plsc.VectorSubcoreMesh(core_axis_name="c", subcore_axis_name="s")


@functools.partial(
    pl.kernel,
    mesh=_mesh,
    out_type=jax.ShapeDtypeStruct((_NUM_ROWS, _D), jnp.float32),
    scratch_types=[
        pltpu.VMEM((_CHUNK,), jnp.int32),
        pltpu.VMEM((_CHUNK, _D), jnp.float32),
        pltpu.SemaphoreType.DMA,
    ],
    compiler_params=pltpu.CompilerParams(use_tc_tiling_on_sc=False),
)
def _gather_rows(idx_hbm, table_hbm, out_hbm, idx_v, rows_v, sem):
    wid = lax.axis_index("s") * _NC + lax.axis_index("c")
    base = wid * _B_PER_W

    def body(i, _):
        off = base + i * _CHUNK
        pltpu.sync_copy(idx_hbm.at[pl.ds(off, _CHUNK)], idx_v)
        pltpu.async_copy(table_hbm.at[idx_v], rows_v, sem).wait()
        pltpu.sync_copy(rows_v, out_hbm.at[pl.ds(off, _CHUNK)])
        return _

    lax.fori_loop(0, _NCHUNK, body, 0)


def kernel(x, table):
    flat_idx = x.reshape(-1)
    out = _gather_rows(flat_idx, table)
    return out.reshape(x.shape[0], -1)


# speedup vs baseline: 1.1899x; 1.0295x over previous
import functools

import jax
import jax.numpy as jnp
from jax import lax
from jax.experimental import pallas as pl
from jax.experimental.pallas import tpu as pltpu
from jax.experimental.pallas import tpu_sc as plsc

_NUM_ROWS = 16384 * 26
_D = 16
_NC = 2
_NS = 16
_NW = _NC * _NS
_B_PER_W = _NUM_ROWS // _NW
_CHUNK = 1664
_NCHUNK = _B_PER_W // _CHUNK
_NBUF = 4

_mesh = plsc.VectorSubcoreMesh(core_axis_name="c", subcore_axis_name="s")


@functools.partial(
    pl.kernel,
    mesh=_mesh,
    out_type=jax.ShapeDtypeStruct((_NUM_ROWS, _D), jnp.float32),
    scratch_types=[
        pltpu.VMEM((_B_PER_W,), jnp.int32),
        [pltpu.VMEM((_CHUNK, _D), jnp.float32) for _ in range(_NBUF)],
        [pltpu.SemaphoreType.DMA for _ in range(_NBUF)],
        [pltpu.SemaphoreType.DMA for _ in range(_NBUF)],
    ],
    compiler_params=pltpu.CompilerParams(use_tc_tiling_on_sc=False),
)
def _gather_rows(idx_hbm, table_hbm, out_hbm, idx_v, rows, g_sems, o_sems):
    wid = lax.axis_index("s") * _NC + lax.axis_index("c")
    base = wid * _B_PER_W

    pltpu.sync_copy(idx_hbm.at[pl.ds(base, _B_PER_W)], idx_v)

    def start_gather(i, b):
        idx_slice = idx_v.at[pl.ds(i * _CHUNK, _CHUNK)]
        return pltpu.async_copy(table_hbm.at[idx_slice], rows[b], g_sems[b])

    gather_dma = [None] * _NCHUNK
    out_dma = [None] * _NCHUNK
    for b in range(_NBUF):
        gather_dma[b] = start_gather(b, b)

    for i in range(_NCHUNK):
        b = i % _NBUF
        gather_dma[i].wait()
        out_dma[i] = pltpu.async_copy(
            rows[b], out_hbm.at[pl.ds(base + i * _CHUNK, _CHUNK)], o_sems[b]
        )
        nxt = i + _NBUF
        if nxt < _NCHUNK:
            out_dma[i].wait()
            gather_dma[nxt] = start_gather(nxt, b)

    for i in range(max(0, _NCHUNK - _NBUF), _NCHUNK):
        out_dma[i].wait()


def kernel(x, table):
    flat_idx = x.reshape(-1)
    out = _gather_rows(flat_idx, table)
    return out.reshape(x.shape[0], -1)
